# Initial kernel scaffold; baseline (speedup 1.0000x reference)
import jax
import jax.numpy as jnp
from jax.experimental import pallas as pl


def kernel(x, edge_index, Wi, bi, W0, a0s, a0d, b0, W1, a1s, a1d, b1, W2, a2s, a2d, b2, W3, a3s, a3d, b3, g0, be0, g1, be1, g2, be2, g3, be3, Wo1, bo1, Wo2, bo2):
    raise NotImplementedError("write your pallas kernel here")


# speedup vs baseline: 30.6382x; 30.6382x over previous
import functools

import jax
import jax.numpy as jnp
from jax import lax
from jax.experimental import pallas as pl
from jax.experimental.pallas import tpu as pltpu
from jax.experimental.pallas import tpu_sc as plsc

_N = 50000
_E = 800000
_H = 64
_BLK = 1000
_GRID = _N // _BLK
_F32 = jnp.float32

_C = 80
_EPT = _E // 16
_NCH = _EPT // _C
_SPAN = 3200


def _lk(v):
    return jnp.where(v >= 0, v, 0.2 * v)


def _mesh():
    return plsc.VectorSubcoreMesh(
        core_axis_name="c", subcore_axis_name="s", num_cores=2, num_subcores=16
    )


def _row_spec(w):
    return pl.BlockSpec((_BLK, w), lambda i: (i, 0))


def _full_spec(r, w):
    return pl.BlockSpec((r, w), lambda i: (0, 0))


def _emit_prologue(hh, as_ref, ad_ref, multi, t_refs, hs_ref, hd_ref, mx_ref, i):
    for k in range(4):
        t_refs[k][...] = hh[:, 16 * k:16 * k + 16]
    if not multi:
        hs = hh @ as_ref[...]
        hd = hh @ ad_ref[...]
    else:
        r_i = lax.broadcasted_iota(jnp.int32, (64, 4), 0)
        c_i = lax.broadcasted_iota(jnp.int32, (64, 4), 1)
        sel = jnp.where(r_i // 16 == c_i, 1.0, 0.0).astype(_F32)
        hs = (hh * as_ref[...]) @ sel
        hd = (hh * ad_ref[...]) @ sel
    hs_ref[...] = hs
    hd_ref[...] = hd
    m = jnp.concatenate(
        [jnp.full((4, 128), jnp.max(hs), _F32),
         jnp.full((4, 128), jnp.max(hd), _F32)], axis=0)

    @pl.when(i == 0)
    def _():
        mx_ref[...] = m

    @pl.when(i != 0)
    def _():
        mx_ref[...] = jnp.maximum(mx_ref[...], m)


def _prologue_shapes(multi):
    ns = 4 if multi else 1
    return ([jax.ShapeDtypeStruct((_N, 16), _F32)] * 4
            + [jax.ShapeDtypeStruct((_N, ns), _F32)] * 2
            + [jax.ShapeDtypeStruct((8, 128), _F32)])


def _prologue_out_specs(multi):
    ns = 4 if multi else 1
    return ([_row_spec(16)] * 4 + [_row_spec(ns)] * 2
            + [pl.BlockSpec((8, 128), lambda i: (0, 0))])


def _tc_pre(x, Wi, bi, W0, a0s, a0d):
    def body(x_ref, wi_ref, bi_ref, w0_ref, as_ref, ad_ref,
             h_ref, t0, t1, t2, t3, hs_ref, hd_ref, mx_ref):
        i = pl.program_id(0)
        h = x_ref[...] @ wi_ref[...] + bi_ref[...]
        h_ref[...] = h
        hh = h @ w0_ref[...]
        _emit_prologue(hh, as_ref, ad_ref, False, (t0, t1, t2, t3),
                       hs_ref, hd_ref, mx_ref, i)

    return pl.pallas_call(
        body,
        grid=(_GRID,),
        in_specs=[_row_spec(24), _full_spec(24, 64), _full_spec(1, 64),
                  _full_spec(64, 64), _full_spec(64, 1), _full_spec(64, 1)],
        out_specs=[_row_spec(64)] + _prologue_out_specs(False),
        out_shape=[jax.ShapeDtypeStruct((_N, 64), _F32)] + _prologue_shapes(False),
    )(x, Wi, bi, W0, a0s, a0d)


def _epilogue(o_refs, s_ref, r_ref, b_ref, g_ref, be_ref, multi_in):
    o = jnp.concatenate([r[...] for r in o_refs], axis=1)
    s = s_ref[...]
    if multi_in:
        srep = jnp.concatenate(
            [jnp.broadcast_to(s[:, k:k + 1], (_BLK, 16)) for k in range(4)],
            axis=1)
        gat = o / (srep + 1e-30)
    else:
        gat = o / (s + 1e-30)
    gat = gat + b_ref[...]
    return _lk(r_ref[...] + gat * g_ref[...] + be_ref[...])


def _tc_mid(multi_in, multi_out, o0, o1, o2, o3, s_arr, r_prev,
            b, gs, be, W, a_s, a_d):
    ns_in = 4 if multi_in else 1

    def body(o0r, o1r, o2r, o3r, s_ref, r_ref, b_ref, g_ref, be_ref,
             w_ref, as_ref, ad_ref,
             h_ref, t0, t1, t2, t3, hs_ref, hd_ref, mx_ref):
        i = pl.program_id(0)
        h = _epilogue((o0r, o1r, o2r, o3r), s_ref, r_ref, b_ref, g_ref,
                      be_ref, multi_in)
        h_ref[...] = h
        hh = h @ w_ref[...]
        _emit_prologue(hh, as_ref, ad_ref, multi_out, (t0, t1, t2, t3),
                       hs_ref, hd_ref, mx_ref, i)

    a_spec = _full_spec(1, 64) if multi_out else _full_spec(64, 1)
    return pl.pallas_call(
        body,
        grid=(_GRID,),
        in_specs=[_row_spec(16)] * 4 + [_row_spec(ns_in), _row_spec(64),
                  _full_spec(1, 64), _full_spec(1, 64), _full_spec(1, 64),
                  _full_spec(64, 64), a_spec, a_spec],
        out_specs=[_row_spec(64)] + _prologue_out_specs(multi_out),
        out_shape=[jax.ShapeDtypeStruct((_N, 64), _F32)] + _prologue_shapes(multi_out),
    )(o0, o1, o2, o3, s_arr, r_prev, b, gs, be, W, a_s, a_d)


def _tc_fin(o0, o1, o2, o3, s_arr, r_prev, b, gs, be, Wo1, bo1, Wo2, bo2, xt):
    def body(o0r, o1r, o2r, o3r, s_ref, r_ref, b_ref, g_ref, be_ref,
             w1_ref, b1_ref, w2_ref, b2_ref, xt_ref, out_ref):
        h = _epilogue((o0r, o1r, o2r, o3r), s_ref, r_ref, b_ref, g_ref,
                      be_ref, True)
        o = _lk(h @ w1_ref[...] + b1_ref[...])
        out_ref[...] = o @ w2_ref[...] + b2_ref[...] + xt_ref[...]

    return pl.pallas_call(
        body,
        grid=(_GRID,),
        in_specs=[_row_spec(16)] * 4 + [_row_spec(4), _row_spec(64),
                  _full_spec(1, 64), _full_spec(1, 64), _full_spec(1, 64),
                  _full_spec(64, 64), _full_spec(1, 64),
                  _full_spec(64, 4), _full_spec(1, 4), _row_spec(4)],
        out_specs=pl.BlockSpec((_BLK, 4), lambda i: (i, 0)),
        out_shape=jax.ShapeDtypeStruct((_N, 4), _F32),
    )(o0, o1, o2, o3, s_arr, r_prev, b, gs, be, Wo1, bo1, Wo2, bo2, xt)


def _zero_buffers(rows0, e_v):
    def zrow(j, acc):
        rows0[j, :] = jnp.zeros((16,), _F32)
        return acc

    lax.fori_loop(0, _C, zrow, 0)
    for j in range(_C // 16):
        e_v[pl.ds(j * 16, 16)] = jnp.zeros((16,), _F32)


def _zero_shared(s_idx, rows0, e_v, row_targets, vec_targets):
    base_n = s_idx * _SPAN
    ncp = jnp.where(s_idx == 15, (_N - 15 * _SPAN) // _C, _SPAN // _C)

    def zcp(j, acc):
        r0 = base_n + j * _C
        for t in row_targets:
            pltpu.sync_copy(rows0, t.at[pl.ds(r0, _C)])
        for t in vec_targets:
            pltpu.sync_copy(e_v, t.at[pl.ds(r0, _C)])
        return acc

    lax.fori_loop(0, ncp, zcp, 0)


def _copy_out(s_idx, pairs):
    base_n = s_idx * _SPAN
    ncp = jnp.where(s_idx == 15, (_N - 15 * _SPAN) // _C, _SPAN // _C)

    def cp(j, acc):
        r0 = base_n + j * _C
        for src, dstr in pairs:
            pltpu.sync_copy(src.at[pl.ds(r0, _C)], dstr.at[pl.ds(r0, _C)])
        return acc

    lax.fori_loop(0, ncp, cp, 0)


def _weight_rows(e_v, rows):
    def wr(i, acc):
        e16 = e_v[pl.ds(i * 16, 16)]
        for j in range(16):
            r = i * 16 + j
            w = e16[j]
            rows[r, :] = rows[r, :] * w
        return acc

    lax.fori_loop(0, _C // 16, wr, 0)


def _exp_chunk(a_v, b_v, e_v, L_v):
    Lvec = L_v[...]

    def ev(i, acc):
        lg = a_v[pl.ds(i * 16, 16)] + b_v[pl.ds(i * 16, 16)]
        lg = jnp.where(lg >= 0, lg, lg * 0.2)
        e_v[pl.ds(i * 16, 16)] = jnp.exp(lg - Lvec)
        return acc

    lax.fori_loop(0, _C // 16, ev, 0)


def _sc_pass(two_e, esrc, edst, t0, t1, t2, t3, hs_list, hd_list, Lv):
    n_s = 4 if two_e else 1
    out_type = ([jax.ShapeDtypeStruct((_N, 16), _F32)] * 4
                + [jax.ShapeDtypeStruct((_N,), _F32)] * n_s)
    scratch = [
        pltpu.VMEM((_C,), jnp.int32),
        pltpu.VMEM((_C,), jnp.int32),
        pltpu.VMEM((_C,), _F32),
        pltpu.VMEM((_C,), _F32),
        pltpu.VMEM((_C,), _F32),
        pltpu.VMEM((_C,), _F32),
        pltpu.VMEM((_C,), _F32),
        pltpu.VMEM((_C,), _F32),
        pltpu.VMEM((_C, 16), _F32),
        pltpu.VMEM((_C, 16), _F32),
        pltpu.VMEM((16,), _F32),
        pltpu.VMEM_SHARED((_N, 16), _F32),
        pltpu.VMEM_SHARED((_N, 16), _F32),
        pltpu.VMEM_SHARED((_N,), _F32),
        pltpu.VMEM_SHARED((_N,), _F32),
        pltpu.SemaphoreType.DMA,
        pltpu.SemaphoreType.DMA,
        pltpu.SemaphoreType.DMA,
        pltpu.SemaphoreType.DMA,
    ]

    @functools.partial(pl.kernel, out_type=out_type, mesh=_mesh(),
                       scratch_types=scratch,
                       compiler_params=pltpu.CompilerParams(
                           needs_layout_passes=False,
                           use_tc_tiling_on_sc=False))
    def k(*refs):
        n_in = 9 + (6 if two_e else 0)
        (src_r, dst_r, t0_r, t1_r, t2_r, t3_r) = refs[:6]
        attn = refs[6:n_in - 1]
        L_r = refs[n_in - 1]
        outs = refs[n_in:n_in + 4 + n_s]
        (src_v, dst_v, hsA_v, hdA_v, hsB_v, hdB_v, e0_v, e1_v,
         rows0, rows1, L_v, accA, accB, s_shA, s_shB,
         sg0, sg1, sg2, sg3) = refs[n_in + 4 + n_s:]
        o_outs = outs[:4]
        s_outs = outs[4:]
        c = lax.axis_index("c")
        s = lax.axis_index("s")
        _zero_buffers(rows0, e0_v)
        _zero_shared(s, rows0, e0_v, [accA, accB],
                     [s_shA, s_shB] if two_e else [s_shA])
        pltpu.sync_copy(L_r, L_v)
        plsc.subcore_barrier()

        def run_core(tA, tB, oA, oB, sA_o, sB_o, hsX, hdX, hsY, hdY, do_s):
            ebase = s * _EPT

            def chunk(g, acc):
                off = ebase + g * _C
                pltpu.sync_copy(src_r.at[pl.ds(off, _C)], src_v)
                pltpu.sync_copy(dst_r.at[pl.ds(off, _C)], dst_v)
                d0 = pltpu.async_copy(tA.at[src_v], rows0, sg0)
                d1 = pltpu.async_copy(tB.at[src_v], rows1, sg1)
                d2 = pltpu.async_copy(hsX.at[src_v], hsA_v, sg2)
                d3 = pltpu.async_copy(hdX.at[dst_v], hdA_v, sg3)
                d2.wait()
                d3.wait()
                _exp_chunk(hsA_v, hdA_v, e0_v, L_v)
                if two_e:
                    d4 = pltpu.async_copy(hsY.at[src_v], hsB_v, sg2)
                    d5 = pltpu.async_copy(hdY.at[dst_v], hdB_v, sg3)
                    d4.wait()
                    d5.wait()
                    _exp_chunk(hsB_v, hdB_v, e1_v, L_v)
                if do_s:
                    pltpu.sync_copy(e0_v, s_shA.at[dst_v], add=True)
                    if two_e:
                        pltpu.sync_copy(e1_v, s_shB.at[dst_v], add=True)
                d0.wait()
                d1.wait()
                _weight_rows(e0_v, rows0)
                _weight_rows(e1_v if two_e else e0_v, rows1)
                pltpu.sync_copy(rows0, accA.at[dst_v], add=True)
                pltpu.sync_copy(rows1, accB.at[dst_v], add=True)
                return acc

            lax.fori_loop(0, _NCH, chunk, 0)
            plsc.subcore_barrier()
            pairs = [(accA, oA), (accB, oB)]
            if do_s:
                pairs.append((s_shA, sA_o))
                if two_e:
                    pairs.append((s_shB, sB_o))
            _copy_out(s, pairs)

        if two_e:
            (hs0, hs1, hs2, hs3, hd0, hd1, hd2, hd3) = attn

            @pl.when(c == 0)
            def _():
                run_core(t0_r, t1_r, o_outs[0], o_outs[1], s_outs[0],
                         s_outs[1], hs0, hd0, hs1, hd1, True)

            @pl.when(c == 1)
            def _():
                run_core(t2_r, t3_r, o_outs[2], o_outs[3], s_outs[2],
                         s_outs[3], hs2, hd2, hs3, hd3, True)
        else:
            (hs_a, hd_a) = attn

            @pl.when(c == 0)
            def _():
                run_core(t0_r, t1_r, o_outs[0], o_outs[1], s_outs[0],
                         None, hs_a, hd_a, hs_a, hd_a, True)

            @pl.when(c == 1)
            def _():
                run_core(t2_r, t3_r, o_outs[2], o_outs[3], None,
                         None, hs_a, hd_a, hs_a, hd_a, False)

    return k(esrc, edst, t0, t1, t2, t3, *hs_list, *hd_list, Lv)


def _lbound(mx):
    t = jnp.max(mx[0:4]) + jnp.max(mx[4:8])
    t = jnp.where(t >= 0, t, 0.2 * t)
    return jnp.full((16,), t, _F32)


def kernel(x, edge_index, Wi, bi, W0, a0s, a0d, b0, W1, a1s, a1d, b1,
           W2, a2s, a2d, b2, W3, a3s, a3d, b3,
           g0, be0, g1, be1, g2, be2, g3, be3, Wo1, bo1, Wo2, bo2):
    esrc, edst = edge_index[0], edge_index[1]
    bn = jnp.sqrt(jnp.asarray(1.0 + 1e-5, _F32))
    r1 = lambda a: a.reshape(1, 64)
    gs = [r1(g / bn) for g in (g0, g1, g2, g3)]
    bes = [r1(b) for b in (be0, be1, be2, be3)]
    bs = [r1(b) for b in (b0, b1, b2, b3)]

    h0, t0, t1, t2, t3, hs, hd, mx = _tc_pre(
        x, Wi, bi.reshape(1, 64), W0, a0s.reshape(64, 1), a0d.reshape(64, 1))
    o0, o1, o2, o3, sv = _sc_pass(
        False, esrc, edst, t0, t1, t2, t3, [hs.reshape(_N)],
        [hd.reshape(_N)], _lbound(mx))

    h1, t0, t1, t2, t3, hs4, hd4, mx = _tc_mid(
        False, True, o0, o1, o2, o3, sv.reshape(_N, 1), h0,
        bs[0], gs[0], bes[0], W1, a1s.reshape(1, 64), a1d.reshape(1, 64))
    o0, o1, o2, o3, s0, s1, s2, s3 = _sc_pass(
        True, esrc, edst, t0, t1, t2, t3,
        [hs4[:, 0], hs4[:, 1], hs4[:, 2], hs4[:, 3]],
        [hd4[:, 0], hd4[:, 1], hd4[:, 2], hd4[:, 3]], _lbound(mx))

    h2, t0, t1, t2, t3, hs, hd, mx = _tc_mid(
        True, False, o0, o1, o2, o3, jnp.stack([s0, s1, s2, s3], axis=1), h1,
        bs[1], gs[1], bes[1], W2, a2s.reshape(64, 1), a2d.reshape(64, 1))
    o0, o1, o2, o3, sv = _sc_pass(
        False, esrc, edst, t0, t1, t2, t3, [hs.reshape(_N)],
        [hd.reshape(_N)], _lbound(mx))

    h3, t0, t1, t2, t3, hs4, hd4, mx = _tc_mid(
        False, True, o0, o1, o2, o3, sv.reshape(_N, 1), h2,
        bs[2], gs[2], bes[2], W3, a3s.reshape(1, 64), a3d.reshape(1, 64))
    o0, o1, o2, o3, s0, s1, s2, s3 = _sc_pass(
        True, esrc, edst, t0, t1, t2, t3,
        [hs4[:, 0], hs4[:, 1], hs4[:, 2], hs4[:, 3]],
        [hd4[:, 0], hd4[:, 1], hd4[:, 2], hd4[:, 3]], _lbound(mx))

    return _tc_fin(o0, o1, o2, o3, jnp.stack([s0, s1, s2, s3], axis=1), h3,
                   bs[3], gs[3], bes[3], Wo1, bo1.reshape(1, 64),
                   Wo2, bo2.reshape(1, 4), x[:, -4:])
